# speedup vs baseline: 2.6901x; 2.6901x over previous
import functools

import jax
import jax.numpy as jnp
from jax import lax
from jax.experimental import pallas as pl

_K = 8
_EPS = 1e-10


def _block_kernel(scores_ref, g_ref, out_ref, *, n_cols):
    s = scores_ref[...] + g_ref[...]
    c = jnp.max(s, axis=1, keepdims=True)
    w = jnp.exp(s - c)
    kh = jnp.zeros_like(w)
    for t in range(_K):
        d = jnp.sum(w, axis=1, keepdims=True)
        oh = w * (1.0 / d)
        kh = kh + oh
        if t + 1 < _K:
            w = w * jnp.maximum(1.0 - oh, _EPS)

    col = lax.broadcasted_iota(jnp.int32, kh.shape, 1)
    out = jnp.zeros_like(kh)
    for _ in range(_K):
        m = jnp.max(kh, axis=1, keepdims=True)
        cand = jnp.where(kh == m, col, n_cols)
        j = jnp.min(cand, axis=1, keepdims=True)
        sel = col == j
        out = jnp.where(sel, 1.0, out)
        kh = jnp.where(sel, -1.0, kh)
    out_ref[...] = out


def kernel(scores, g):
    b, n = scores.shape
    rows = 8
    grid = (b // rows,)
    spec = pl.BlockSpec((rows, n), lambda i: (i, 0))
    return pl.pallas_call(
        functools.partial(_block_kernel, n_cols=n),
        grid=grid,
        in_specs=[spec, spec],
        out_specs=spec,
        out_shape=jax.ShapeDtypeStruct((b, n), jnp.float32),
    )(scores, g)


# speedup vs baseline: 4.5824x; 1.7034x over previous
import functools

import jax
import jax.numpy as jnp
from jax import lax
from jax.experimental import pallas as pl

_K = 8
_EPS = 1e-10
_LANES = 128


def _block_kernel(scores_ref, g_ref, out_ref, *, n_cols):
    s = scores_ref[...] + g_ref[...]
    c = jnp.max(s, axis=1, keepdims=True)
    w = jnp.exp(s - c)
    kh = jnp.zeros_like(w)
    for t in range(_K):
        d = jnp.sum(w, axis=1, keepdims=True)
        oh = w * (1.0 / d)
        kh = kh + oh
        if t + 1 < _K:
            w = w * jnp.maximum(1.0 - oh, _EPS)

    rows = kh.shape[0]
    n_chunks = n_cols // _LANES

    neg = jnp.full((rows, _LANES), -jnp.inf, jnp.float32)
    accs = [neg] * _K
    for k in range(n_chunks):
        x = kh[:, k * _LANES:(k + 1) * _LANES]
        for j in range(_K):
            hi = jnp.maximum(accs[j], x)
            x = jnp.minimum(accs[j], x)
            accs[j] = hi

    kcum = jnp.zeros((rows, 1), jnp.float32)
    tval = jnp.full((rows, 1), -jnp.inf, jnp.float32)
    work = list(accs)
    for t in range(_K):
        m = work[0]
        for j in range(1, _K):
            m = jnp.maximum(m, work[j])
        v = jnp.max(m, axis=1, keepdims=True)
        cnt = jnp.zeros((rows, 1), jnp.float32)
        for j in range(_K):
            cnt = cnt + jnp.sum((work[j] == v).astype(jnp.float32),
                                axis=1, keepdims=True)
        tval = jnp.where(kcum < 8.0, v, tval)
        kcum = kcum + cnt
        if t + 1 < _K:
            work = [jnp.where(wj == v, -jnp.inf, wj) for wj in work]

    ge = kh >= tval
    n_ge = jnp.sum(ge.astype(jnp.float32), axis=1, keepdims=True)
    exact = jnp.all(n_ge == 8.0)

    @pl.when(exact)
    def _():
        out_ref[...] = ge.astype(jnp.float32)

    @pl.when(jnp.logical_not(exact))
    def _():
        col = lax.broadcasted_iota(jnp.int32, kh.shape, 1)
        gt = kh > tval
        need = 8.0 - jnp.sum(gt.astype(jnp.float32), axis=1, keepdims=True)
        base = gt
        last = jnp.full((rows, 1), -1, jnp.int32)
        for t in range(_K):
            cand = jnp.where((kh == tval) & (col > last), col, n_cols)
            j = jnp.min(cand, axis=1, keepdims=True)
            take = (float(t) < need) & (j < n_cols)
            base = base | (take & (col == j))
            last = jnp.where(take, j, last)
        out_ref[...] = base.astype(jnp.float32)


def kernel(scores, g):
    b, n = scores.shape
    rows = 8
    grid = (b // rows,)
    spec = pl.BlockSpec((rows, n), lambda i: (i, 0))
    return pl.pallas_call(
        functools.partial(_block_kernel, n_cols=n),
        grid=grid,
        in_specs=[spec, spec],
        out_specs=spec,
        out_shape=jax.ShapeDtypeStruct((b, n), jnp.float32),
    )(scores, g)
